# baseline (speedup 1.0000x reference)
import jax
import jax.numpy as jnp
from jax.experimental import pallas as pl
from jax.experimental.pallas import tpu as pltpu


def _round_up(v, m):
    return ((v + m - 1) // m) * m


def _make_kernel(nk, tm):
    def _dense_kernel(x_ref, w_ref, b_ref, o_ref, acc_ref):
        k = pl.program_id(0)
        i = pl.program_id(1)
        part = jnp.dot(x_ref[...], w_ref[...],
                       preferred_element_type=jnp.float32)
        rows = pl.ds(i * tm, tm)
        if nk == 1:
            o_ref[...] = (part + b_ref[...].astype(jnp.float32)
                          ).astype(o_ref.dtype)
            return

        @pl.when(k == 0)
        def _init():
            acc_ref[rows, :] = part

        @pl.when(jnp.logical_and(k > 0, k < nk - 1))
        def _accum():
            acc_ref[rows, :] += part

        @pl.when(k == nk - 1)
        def _final():
            o_ref[...] = (acc_ref[rows, :] + part
                          + b_ref[...].astype(jnp.float32)).astype(o_ref.dtype)

    return _dense_kernel


def kernel(x, w_kn, b):
    in_dim, out_dim = w_kn.shape
    orig_shape = x.shape
    out_dtype = x.dtype

    x2 = x.reshape(-1, in_dim)
    m = x2.shape[0]

    k_p = _round_up(in_dim, 128)
    n_p = _round_up(out_dim, 128)
    w_p = w_kn
    if (k_p, n_p) != (in_dim, out_dim):
        w_p = jnp.pad(w_kn, ((0, k_p - in_dim), (0, n_p - out_dim)))
    b_p = b
    if b.shape != (1, n_p):
        b_p = jnp.pad(b.reshape(1, -1), ((0, 0), (0, n_p - b.size)))

    tm = min(512, _round_up(m, 8))
    m_p = _round_up(m, tm)
    x_p = x2
    if (m_p, k_p) != (m, in_dim):
        x_p = jnp.pad(x2, ((0, m_p - m), (0, k_p - in_dim)))

    tk = 512 if k_p % 512 == 0 else k_p
    nk = k_p // tk
    ni = m_p // tm
    grid = (nk, ni)

    x_item = jnp.dtype(x_p.dtype).itemsize
    o_item = jnp.dtype(out_dtype).itemsize
    cost = pl.CostEstimate(
        flops=2 * m_p * k_p * n_p,
        transcendentals=0,
        bytes_accessed=(m_p * k_p * x_item + k_p * n_p * 4
                        + n_p * 4 + m_p * n_p * o_item),
    )

    last_k = nk - 1

    out_p = pl.pallas_call(
        _make_kernel(nk, tm),
        out_shape=jax.ShapeDtypeStruct((m_p, n_p), out_dtype),
        grid=grid,
        in_specs=[
            pl.BlockSpec((tm, tk), lambda k, i: (i, k)),
            pl.BlockSpec((tk, n_p), lambda k, i: (k, 0)),
            pl.BlockSpec((1, n_p), lambda k, i: (0, 0)),
        ],
        out_specs=pl.BlockSpec(
            (tm, n_p), lambda k, i: (jnp.where(k == last_k, i, 0), 0)),
        scratch_shapes=[pltpu.VMEM((m_p, n_p), jnp.float32)],
        compiler_params=pltpu.CompilerParams(
            dimension_semantics=("arbitrary", "arbitrary"),
            vmem_limit_bytes=60 * 1024 * 1024,
        ),
        cost_estimate=cost,
    )(x_p, w_p, b_p)

    out = out_p[:m, :out_dim]
    return out.reshape(orig_shape[:-1] + (out_dim,))


# speedup vs baseline: 1.2134x; 1.2134x over previous
import jax
import jax.numpy as jnp
from jax.experimental import pallas as pl
from jax.experimental.pallas import tpu as pltpu


def _round_up(v, m):
    return ((v + m - 1) // m) * m


def _make_kernel(nk, tm):
    def _dense_kernel(x_ref, w_ref, b_ref, o_ref, acc_ref):
        k = pl.program_id(0)
        i = pl.program_id(1)
        part = jnp.dot(x_ref[...], w_ref[...],
                       preferred_element_type=jnp.float32)
        rows = pl.ds(i * tm, tm)
        if nk == 1:
            o_ref[...] = (part + b_ref[...].astype(jnp.float32)
                          ).astype(o_ref.dtype)
            return

        @pl.when(k == 0)
        def _init():
            acc_ref[rows, :] = part.astype(acc_ref.dtype)

        @pl.when(jnp.logical_and(k > 0, k < nk - 1))
        def _accum():
            acc_ref[rows, :] = (acc_ref[rows, :].astype(jnp.float32)
                                + part).astype(acc_ref.dtype)

        @pl.when(k == nk - 1)
        def _final():
            o_ref[...] = (acc_ref[rows, :].astype(jnp.float32) + part
                          + b_ref[...].astype(jnp.float32)).astype(o_ref.dtype)

    return _dense_kernel


def kernel(x, w_kn, b):
    in_dim, out_dim = w_kn.shape
    orig_shape = x.shape
    out_dtype = x.dtype

    x2 = x.reshape(-1, in_dim)
    m = x2.shape[0]

    k_p = _round_up(in_dim, 128)
    n_p = _round_up(out_dim, 128)
    w_p = w_kn
    if (k_p, n_p) != (in_dim, out_dim):
        w_p = jnp.pad(w_kn, ((0, k_p - in_dim), (0, n_p - out_dim)))
    b_p = b
    if b.shape != (1, n_p):
        b_p = jnp.pad(b.reshape(1, -1), ((0, 0), (0, n_p - b.size)))

    tm = min(512, _round_up(m, 8))
    m_p = _round_up(m, tm)
    x_p = x2
    if (m_p, k_p) != (m, in_dim):
        x_p = jnp.pad(x2, ((0, m_p - m), (0, k_p - in_dim)))

    tk = 1024 if k_p % 1024 == 0 else k_p
    nk = k_p // tk
    ni = m_p // tm
    grid = (nk, ni)

    x_item = jnp.dtype(x_p.dtype).itemsize
    o_item = jnp.dtype(out_dtype).itemsize
    cost = pl.CostEstimate(
        flops=2 * m_p * k_p * n_p,
        transcendentals=0,
        bytes_accessed=(m_p * k_p * x_item + k_p * n_p * 4
                        + n_p * 4 + m_p * n_p * o_item),
    )

    last_k = nk - 1

    out_p = pl.pallas_call(
        _make_kernel(nk, tm),
        out_shape=jax.ShapeDtypeStruct((m_p, n_p), out_dtype),
        grid=grid,
        in_specs=[
            pl.BlockSpec((tm, tk), lambda k, i: (i, k)),
            pl.BlockSpec((tk, n_p), lambda k, i: (k, 0)),
            pl.BlockSpec((1, n_p), lambda k, i: (0, 0)),
        ],
        out_specs=pl.BlockSpec(
            (tm, n_p), lambda k, i: (jnp.where(k == last_k, i, 0), 0)),
        scratch_shapes=[pltpu.VMEM((m_p, n_p), jnp.bfloat16)],
        compiler_params=pltpu.CompilerParams(
            dimension_semantics=("arbitrary", "arbitrary"),
            vmem_limit_bytes=60 * 1024 * 1024,
        ),
        cost_estimate=cost,
    )(x_p, w_p, b_p)

    out = out_p[:m, :out_dim]
    return out.reshape(orig_shape[:-1] + (out_dim,))
